# baseline (speedup 1.0000x reference)
import jax
import jax.numpy as jnp
from jax.experimental import pallas as pl
from jax.experimental.pallas import tpu as pltpu

N = 4096
DO = 64
BK_ENC = 512
BK_DEC = 256


def _dot(a, b):
    return jax.lax.dot_general(
        a, b, (((1,), (0,)), ((), ())),
        preferred_element_type=jnp.float32)


def _enc_body(params_ref, a_s_ref, a_f_ref, feat_ref, w_enc_ref,
              emb_ref, y_ref):
    k = pl.program_id(0)

    @pl.when(k == 0)
    def _init():
        y = _dot(feat_ref[...], w_enc_ref[...])
        y_ref[...] = y
        bias = params_ref[2] * jnp.sum(y, axis=0, keepdims=True)
        emb_ref[...] = jnp.broadcast_to(bias, (N, DO))

    yk = y_ref[pl.ds(k * BK_ENC, BK_ENC), :]
    p_s = _dot(a_s_ref[...], yk)
    p_f = _dot(a_f_ref[...], yk)
    emb_ref[...] += params_ref[0] * p_s + params_ref[1] * p_f


def _dec_body(emb1_ref, emb2_ref, emb3_ref,
              w_fc1_ref, b_fc1_ref, w_fc2_ref, b_fc2_ref,
              w_dec1_ref, w_dec2_ref, w_dec3_ref,
              a_s1_ref, a_s2_ref, a_s3_ref,
              comb_ref, rec1_ref, rec2_ref, rec3_ref,
              d1_ref, d2_ref, d3_ref):
    k = pl.program_id(0)

    @pl.when(k == 0)
    def _init():
        t = (_dot(emb1_ref[...], w_fc1_ref[0:DO, :])
             + _dot(emb2_ref[...], w_fc1_ref[DO:2 * DO, :])
             + _dot(emb3_ref[...], w_fc1_ref[2 * DO:3 * DO, :])
             + b_fc1_ref[...])
        comb = _dot(t, w_fc2_ref[...]) + b_fc2_ref[...]
        comb_ref[...] = comb
        d1_ref[...] = _dot(comb, w_dec1_ref[...])
        d2_ref[...] = _dot(comb, w_dec2_ref[...])
        d3_ref[...] = _dot(comb, w_dec3_ref[...])
        rec1_ref[...] = jnp.zeros(rec1_ref.shape, rec1_ref.dtype)
        rec2_ref[...] = jnp.zeros(rec2_ref.shape, rec2_ref.dtype)
        rec3_ref[...] = jnp.zeros(rec3_ref.shape, rec3_ref.dtype)

    rec1_ref[...] += _dot(a_s1_ref[...], d1_ref[pl.ds(k * BK_DEC, BK_DEC), :])
    rec2_ref[...] += _dot(a_s2_ref[...], d2_ref[pl.ds(k * BK_DEC, BK_DEC), :])
    rec3_ref[...] += _dot(a_s3_ref[...], d3_ref[pl.ds(k * BK_DEC, BK_DEC), :])


def _stream_spec(bk):
    return pl.BlockSpec((N, bk), lambda k: (0, k))


def _full_spec(shape):
    return pl.BlockSpec(shape, lambda k: tuple(0 for _ in shape))


def _encode_one(params, a_s, a_f, feat, w_enc, d_in):
    return pl.pallas_call(
        _enc_body,
        grid=(N // BK_ENC,),
        in_specs=[
            pl.BlockSpec(memory_space=pltpu.SMEM),
            _stream_spec(BK_ENC),
            _stream_spec(BK_ENC),
            _full_spec((N, d_in)),
            _full_spec((d_in, DO)),
        ],
        out_specs=_full_spec((N, DO)),
        out_shape=jax.ShapeDtypeStruct((N, DO), jnp.float32),
        scratch_shapes=[pltpu.VMEM((N, DO), jnp.float32)],
        compiler_params=pltpu.CompilerParams(
            dimension_semantics=("arbitrary",)),
    )(params, a_s, a_f, feat, w_enc)


def _decode(emb1, emb2, emb3, w_fc1, b_fc1, w_fc2, b_fc2,
            w_dec1, w_dec2, w_dec3, a_s1, a_s2, a_s3, d1, d2, d3):
    return pl.pallas_call(
        _dec_body,
        grid=(N // BK_DEC,),
        in_specs=[
            _full_spec((N, DO)), _full_spec((N, DO)), _full_spec((N, DO)),
            _full_spec((3 * DO, DO)), _full_spec((1, DO)),
            _full_spec((DO, DO)), _full_spec((1, DO)),
            _full_spec((DO, d1)), _full_spec((DO, d2)), _full_spec((DO, d3)),
            _stream_spec(BK_DEC), _stream_spec(BK_DEC), _stream_spec(BK_DEC),
        ],
        out_specs=[
            _full_spec((N, DO)),
            _full_spec((N, d1)), _full_spec((N, d2)), _full_spec((N, d3)),
        ],
        out_shape=[
            jax.ShapeDtypeStruct((N, DO), jnp.float32),
            jax.ShapeDtypeStruct((N, d1), jnp.float32),
            jax.ShapeDtypeStruct((N, d2), jnp.float32),
            jax.ShapeDtypeStruct((N, d3), jnp.float32),
        ],
        scratch_shapes=[
            pltpu.VMEM((N, d1), jnp.float32),
            pltpu.VMEM((N, d2), jnp.float32),
            pltpu.VMEM((N, d3), jnp.float32),
        ],
        compiler_params=pltpu.CompilerParams(
            dimension_semantics=("arbitrary",)),
    )(emb1, emb2, emb3, w_fc1, b_fc1, w_fc2, b_fc2,
      w_dec1, w_dec2, w_dec3, a_s1, a_s2, a_s3)


def kernel(features_omics1, features_omics2, features_omics3,
           adj_spatial_omics1, adj_feature_omics1,
           adj_spatial_omics2, adj_feature_omics2,
           adj_spatial_omics3, adj_feature_omics3,
           W_conv1, b_conv1, W_conv2, b_conv2, W_conv3, b_conv3,
           W_enc1, W_enc2, W_enc3, W_dec1, W_dec2, W_dec3,
           W_fc1, b_fc1, W_fc2, b_fc2):
    p1 = jnp.concatenate([W_conv1, b_conv1])
    p2 = jnp.concatenate([W_conv2, b_conv2])
    p3 = jnp.concatenate([W_conv3, b_conv3])

    emb1 = _encode_one(p1, adj_spatial_omics1, adj_feature_omics1,
                       features_omics1, W_enc1, features_omics1.shape[1])
    emb2 = _encode_one(p2, adj_spatial_omics2, adj_feature_omics2,
                       features_omics2, W_enc2, features_omics2.shape[1])
    emb3 = _encode_one(p3, adj_spatial_omics3, adj_feature_omics3,
                       features_omics3, W_enc3, features_omics3.shape[1])

    comb, rec1, rec2, rec3 = _decode(
        emb1, emb2, emb3,
        W_fc1, b_fc1.reshape(1, DO), W_fc2, b_fc2.reshape(1, DO),
        W_dec1, W_dec2, W_dec3,
        adj_spatial_omics1, adj_spatial_omics2, adj_spatial_omics3,
        W_dec1.shape[1], W_dec2.shape[1], W_dec3.shape[1])

    return (emb1, emb2, emb3, comb, rec1, rec2, rec3)


# speedup vs baseline: 1.0054x; 1.0054x over previous
import jax
import jax.numpy as jnp
from jax.experimental import pallas as pl
from jax.experimental.pallas import tpu as pltpu

N = 4096
DO = 64
BM_ENC = 512
BM_DEC = 256


def _dot(a, b):
    return jax.lax.dot_general(
        a, b, (((1,), (0,)), ((), ())),
        preferred_element_type=jnp.float32)


def _enc_body(params_ref, a_s_ref, a_f_ref, feat_ref, w_enc_ref,
              emb_ref, y_ref, bias_ref):
    i = pl.program_id(0)

    @pl.when(i == 0)
    def _init():
        y = _dot(feat_ref[...], w_enc_ref[...])
        y_ref[...] = y
        bias = params_ref[2] * jnp.sum(y, axis=0, keepdims=True)
        bias_ref[...] = jnp.broadcast_to(bias, (8, DO))

    y = y_ref[...]
    emb_ref[...] = (params_ref[0] * _dot(a_s_ref[...], y)
                    + params_ref[1] * _dot(a_f_ref[...], y)
                    + bias_ref[0:1, :])


def _dec_body(emb1_ref, emb2_ref, emb3_ref,
              w_fc1_ref, b_fc1_ref, w_fc2_ref, b_fc2_ref,
              w_dec1_ref, w_dec2_ref, w_dec3_ref,
              a_s1_ref, a_s2_ref, a_s3_ref,
              comb_ref, rec1_ref, rec2_ref, rec3_ref,
              d1_ref, d2_ref, d3_ref):
    i = pl.program_id(0)

    @pl.when(i == 0)
    def _init():
        t = (_dot(emb1_ref[...], w_fc1_ref[0:DO, :])
             + _dot(emb2_ref[...], w_fc1_ref[DO:2 * DO, :])
             + _dot(emb3_ref[...], w_fc1_ref[2 * DO:3 * DO, :])
             + b_fc1_ref[...])
        comb = _dot(t, w_fc2_ref[...]) + b_fc2_ref[...]
        comb_ref[...] = comb
        d1_ref[...] = _dot(comb, w_dec1_ref[...])
        d2_ref[...] = _dot(comb, w_dec2_ref[...])
        d3_ref[...] = _dot(comb, w_dec3_ref[...])

    rec1_ref[...] = _dot(a_s1_ref[...], d1_ref[...])
    rec2_ref[...] = _dot(a_s2_ref[...], d2_ref[...])
    rec3_ref[...] = _dot(a_s3_ref[...], d3_ref[...])


def _row_spec(bm, ncols):
    return pl.BlockSpec((bm, ncols), lambda i: (i, 0))


def _full_spec(shape):
    return pl.BlockSpec(shape, lambda i: tuple(0 for _ in shape))


def _encode_one(params, a_s, a_f, feat, w_enc, d_in):
    return pl.pallas_call(
        _enc_body,
        grid=(N // BM_ENC,),
        in_specs=[
            pl.BlockSpec(memory_space=pltpu.SMEM),
            _row_spec(BM_ENC, N),
            _row_spec(BM_ENC, N),
            _full_spec((N, d_in)),
            _full_spec((d_in, DO)),
        ],
        out_specs=_row_spec(BM_ENC, DO),
        out_shape=jax.ShapeDtypeStruct((N, DO), jnp.float32),
        scratch_shapes=[pltpu.VMEM((N, DO), jnp.float32),
                        pltpu.VMEM((8, DO), jnp.float32)],
        compiler_params=pltpu.CompilerParams(
            dimension_semantics=("arbitrary",)),
    )(params, a_s, a_f, feat, w_enc)


def _decode(emb1, emb2, emb3, w_fc1, b_fc1, w_fc2, b_fc2,
            w_dec1, w_dec2, w_dec3, a_s1, a_s2, a_s3, d1, d2, d3):
    return pl.pallas_call(
        _dec_body,
        grid=(N // BM_DEC,),
        in_specs=[
            _full_spec((N, DO)), _full_spec((N, DO)), _full_spec((N, DO)),
            _full_spec((3 * DO, DO)), _full_spec((1, DO)),
            _full_spec((DO, DO)), _full_spec((1, DO)),
            _full_spec((DO, d1)), _full_spec((DO, d2)), _full_spec((DO, d3)),
            _row_spec(BM_DEC, N), _row_spec(BM_DEC, N), _row_spec(BM_DEC, N),
        ],
        out_specs=[
            _full_spec((N, DO)),
            _row_spec(BM_DEC, d1), _row_spec(BM_DEC, d2), _row_spec(BM_DEC, d3),
        ],
        out_shape=[
            jax.ShapeDtypeStruct((N, DO), jnp.float32),
            jax.ShapeDtypeStruct((N, d1), jnp.float32),
            jax.ShapeDtypeStruct((N, d2), jnp.float32),
            jax.ShapeDtypeStruct((N, d3), jnp.float32),
        ],
        scratch_shapes=[
            pltpu.VMEM((N, d1), jnp.float32),
            pltpu.VMEM((N, d2), jnp.float32),
            pltpu.VMEM((N, d3), jnp.float32),
        ],
        compiler_params=pltpu.CompilerParams(
            dimension_semantics=("arbitrary",)),
    )(emb1, emb2, emb3, w_fc1, b_fc1, w_fc2, b_fc2,
      w_dec1, w_dec2, w_dec3, a_s1, a_s2, a_s3)


def kernel(features_omics1, features_omics2, features_omics3,
           adj_spatial_omics1, adj_feature_omics1,
           adj_spatial_omics2, adj_feature_omics2,
           adj_spatial_omics3, adj_feature_omics3,
           W_conv1, b_conv1, W_conv2, b_conv2, W_conv3, b_conv3,
           W_enc1, W_enc2, W_enc3, W_dec1, W_dec2, W_dec3,
           W_fc1, b_fc1, W_fc2, b_fc2):
    p1 = jnp.concatenate([W_conv1, b_conv1])
    p2 = jnp.concatenate([W_conv2, b_conv2])
    p3 = jnp.concatenate([W_conv3, b_conv3])

    emb1 = _encode_one(p1, adj_spatial_omics1, adj_feature_omics1,
                       features_omics1, W_enc1, features_omics1.shape[1])
    emb2 = _encode_one(p2, adj_spatial_omics2, adj_feature_omics2,
                       features_omics2, W_enc2, features_omics2.shape[1])
    emb3 = _encode_one(p3, adj_spatial_omics3, adj_feature_omics3,
                       features_omics3, W_enc3, features_omics3.shape[1])

    comb, rec1, rec2, rec3 = _decode(
        emb1, emb2, emb3,
        W_fc1, b_fc1.reshape(1, DO), W_fc2, b_fc2.reshape(1, DO),
        W_dec1, W_dec2, W_dec3,
        adj_spatial_omics1, adj_spatial_omics2, adj_spatial_omics3,
        W_dec1.shape[1], W_dec2.shape[1], W_dec3.shape[1])

    return (emb1, emb2, emb3, comb, rec1, rec2, rec3)


# speedup vs baseline: 1.0281x; 1.0226x over previous
import jax
import jax.numpy as jnp
from jax.experimental import pallas as pl
from jax.experimental.pallas import tpu as pltpu

N = 4096
DO = 64
BM_ENC = 256
BM_DEC = 256


def _dot(a, b):
    return jax.lax.dot_general(
        a, b, (((1,), (0,)), ((), ())),
        preferred_element_type=jnp.float32)


def _enc_body(params_ref, a_s_ref, a_f_ref, feat_ref, w_enc_ref,
              emb_ref, y_ref, bias_ref):
    i = pl.program_id(0)

    @pl.when(i == 0)
    def _init():
        y = _dot(feat_ref[...], w_enc_ref[...])
        y_ref[...] = y
        bias = params_ref[2] * jnp.sum(y, axis=0, keepdims=True)
        bias_ref[...] = jnp.broadcast_to(bias, (8, DO))

    y = y_ref[...]
    emb_ref[...] = (params_ref[0] * _dot(a_s_ref[...], y)
                    + params_ref[1] * _dot(a_f_ref[...], y)
                    + bias_ref[0:1, :])


def _dec_body(emb1_ref, emb2_ref, emb3_ref,
              w_fc1_ref, b_fc1_ref, w_fc2_ref, b_fc2_ref,
              w_dec1_ref, w_dec2_ref, w_dec3_ref,
              a_s1_ref, a_s2_ref, a_s3_ref,
              comb_ref, rec1_ref, rec2_ref, rec3_ref,
              d1_ref, d2_ref, d3_ref):
    i = pl.program_id(0)

    @pl.when(i == 0)
    def _init():
        t = (_dot(emb1_ref[...], w_fc1_ref[0:DO, :])
             + _dot(emb2_ref[...], w_fc1_ref[DO:2 * DO, :])
             + _dot(emb3_ref[...], w_fc1_ref[2 * DO:3 * DO, :])
             + b_fc1_ref[...])
        comb = _dot(t, w_fc2_ref[...]) + b_fc2_ref[...]
        comb_ref[...] = comb
        d1_ref[...] = _dot(comb, w_dec1_ref[...])
        d2_ref[...] = _dot(comb, w_dec2_ref[...])
        d3_ref[...] = _dot(comb, w_dec3_ref[...])

    rec1_ref[...] = _dot(a_s1_ref[...], d1_ref[...])
    rec2_ref[...] = _dot(a_s2_ref[...], d2_ref[...])
    rec3_ref[...] = _dot(a_s3_ref[...], d3_ref[...])


def _row_spec(bm, ncols):
    return pl.BlockSpec((bm, ncols), lambda i: (i, 0))


def _full_spec(shape):
    return pl.BlockSpec(shape, lambda i: tuple(0 for _ in shape))


def _encode_one(params, a_s, a_f, feat, w_enc, d_in):
    return pl.pallas_call(
        _enc_body,
        grid=(N // BM_ENC,),
        in_specs=[
            pl.BlockSpec(memory_space=pltpu.SMEM),
            _row_spec(BM_ENC, N),
            _row_spec(BM_ENC, N),
            _full_spec((N, d_in)),
            _full_spec((d_in, DO)),
        ],
        out_specs=_row_spec(BM_ENC, DO),
        out_shape=jax.ShapeDtypeStruct((N, DO), jnp.float32),
        scratch_shapes=[pltpu.VMEM((N, DO), jnp.float32),
                        pltpu.VMEM((8, DO), jnp.float32)],
        compiler_params=pltpu.CompilerParams(
            dimension_semantics=("arbitrary",)),
    )(params, a_s, a_f, feat, w_enc)


def _decode(emb1, emb2, emb3, w_fc1, b_fc1, w_fc2, b_fc2,
            w_dec1, w_dec2, w_dec3, a_s1, a_s2, a_s3, d1, d2, d3):
    return pl.pallas_call(
        _dec_body,
        grid=(N // BM_DEC,),
        in_specs=[
            _full_spec((N, DO)), _full_spec((N, DO)), _full_spec((N, DO)),
            _full_spec((3 * DO, DO)), _full_spec((1, DO)),
            _full_spec((DO, DO)), _full_spec((1, DO)),
            _full_spec((DO, d1)), _full_spec((DO, d2)), _full_spec((DO, d3)),
            _row_spec(BM_DEC, N), _row_spec(BM_DEC, N), _row_spec(BM_DEC, N),
        ],
        out_specs=[
            _full_spec((N, DO)),
            _row_spec(BM_DEC, d1), _row_spec(BM_DEC, d2), _row_spec(BM_DEC, d3),
        ],
        out_shape=[
            jax.ShapeDtypeStruct((N, DO), jnp.float32),
            jax.ShapeDtypeStruct((N, d1), jnp.float32),
            jax.ShapeDtypeStruct((N, d2), jnp.float32),
            jax.ShapeDtypeStruct((N, d3), jnp.float32),
        ],
        scratch_shapes=[
            pltpu.VMEM((N, d1), jnp.float32),
            pltpu.VMEM((N, d2), jnp.float32),
            pltpu.VMEM((N, d3), jnp.float32),
        ],
        compiler_params=pltpu.CompilerParams(
            dimension_semantics=("arbitrary",)),
    )(emb1, emb2, emb3, w_fc1, b_fc1, w_fc2, b_fc2,
      w_dec1, w_dec2, w_dec3, a_s1, a_s2, a_s3)


def kernel(features_omics1, features_omics2, features_omics3,
           adj_spatial_omics1, adj_feature_omics1,
           adj_spatial_omics2, adj_feature_omics2,
           adj_spatial_omics3, adj_feature_omics3,
           W_conv1, b_conv1, W_conv2, b_conv2, W_conv3, b_conv3,
           W_enc1, W_enc2, W_enc3, W_dec1, W_dec2, W_dec3,
           W_fc1, b_fc1, W_fc2, b_fc2):
    p1 = jnp.concatenate([W_conv1, b_conv1])
    p2 = jnp.concatenate([W_conv2, b_conv2])
    p3 = jnp.concatenate([W_conv3, b_conv3])

    emb1 = _encode_one(p1, adj_spatial_omics1, adj_feature_omics1,
                       features_omics1, W_enc1, features_omics1.shape[1])
    emb2 = _encode_one(p2, adj_spatial_omics2, adj_feature_omics2,
                       features_omics2, W_enc2, features_omics2.shape[1])
    emb3 = _encode_one(p3, adj_spatial_omics3, adj_feature_omics3,
                       features_omics3, W_enc3, features_omics3.shape[1])

    comb, rec1, rec2, rec3 = _decode(
        emb1, emb2, emb3,
        W_fc1, b_fc1.reshape(1, DO), W_fc2, b_fc2.reshape(1, DO),
        W_dec1, W_dec2, W_dec3,
        adj_spatial_omics1, adj_spatial_omics2, adj_spatial_omics3,
        W_dec1.shape[1], W_dec2.shape[1], W_dec3.shape[1])

    return (emb1, emb2, emb3, comb, rec1, rec2, rec3)
